# baseline (speedup 1.0000x reference)
import functools

import jax
import jax.numpy as jnp
from jax import lax
from jax.experimental import pallas as pl
from jax.experimental.pallas import tpu as pltpu
from jax.experimental.pallas import tpu_sc as plsc

NC = 2
NS = 16
NW = NC * NS
L = 16

B = 16384
D = 32
NIDS = 1000000
BPW = B // NW
CHUNK = 128
NCHUNK = BPW // CHUNK
GROUPS = BPW // L
KPC = CHUNK // L


def _mf_body(uid_hbm, iid_hbm, uemb_hbm, ub16_hbm, iemb_hbm, ib16_hbm,
             out_hbm, uid_v, iid_v, uhi, ihi, ulo, ilo,
             urows, irows, ub, ib, mt, out_v, sem):
    wid = lax.axis_index("s") * NC + lax.axis_index("c")
    base = wid * BPW

    for c in range(NCHUNK):
        pltpu.sync_copy(uid_hbm.at[pl.ds(base + c * CHUNK, CHUNK)], uid_v.at[c])
        pltpu.sync_copy(iid_hbm.at[pl.ds(base + c * CHUNK, CHUNK)], iid_v.at[c])

    for k in range(GROUPS):
        c, off = k // KPC, (k % KPC) * L
        u = uid_v[c, pl.ds(off, L)]
        i = iid_v[c, pl.ds(off, L)]
        uhi[c, pl.ds(off, L)] = u >> 4
        ihi[c, pl.ds(off, L)] = i >> 4
        ulo[pl.ds(k * L, L)] = u & 15
        ilo[pl.ds(k * L, L)] = i & 15

    copies = []
    for c in range(NCHUNK):
        sl = pl.ds(c * CHUNK, CHUNK)
        copies.append(pltpu.async_copy(uemb_hbm.at[uid_v.at[c]], urows.at[sl], sem))
        copies.append(pltpu.async_copy(iemb_hbm.at[iid_v.at[c]], irows.at[sl], sem))
        copies.append(pltpu.async_copy(ub16_hbm.at[uhi.at[c]], ub.at[sl], sem))
        copies.append(pltpu.async_copy(ib16_hbm.at[ihi.at[c]], ib.at[sl], sem))
    for cp in copies:
        cp.wait()

    lanes = lax.iota(jnp.int32, L)

    def group(g, carry):
        r0 = pl.multiple_of(g * L, L)
        for r in range(L):
            row = r0 + r
            p0 = urows[row, pl.ds(0, L)]
            p1 = urows[row, pl.ds(L, L)]
            q0 = irows[row, pl.ds(0, L)]
            q1 = irows[row, pl.ds(L, L)]
            a = p0 * q0 + p1 * q1
            plsc.store_scatter(mt, [lanes, jnp.full((L,), r, jnp.int32)], a)
        rows16 = r0 + lanes
        acc = (plsc.load_gather(ub, [rows16, ulo[pl.ds(r0, L)]]) +
               plsc.load_gather(ib, [rows16, ilo[pl.ds(r0, L)]]))
        for j in range(L):
            acc = acc + mt[j, pl.ds(0, L)]
        out_v[pl.ds(r0, L)] = acc
        return carry

    lax.fori_loop(0, GROUPS, group, 0)
    pltpu.sync_copy(out_v, out_hbm.at[pl.ds(base, BPW)])


_mf_kernel = functools.partial(
    pl.kernel,
    out_type=jax.ShapeDtypeStruct((B,), jnp.float32),
    mesh=plsc.VectorSubcoreMesh(
        core_axis_name="c", subcore_axis_name="s",
        num_cores=NC, num_subcores=NS),
    scratch_types=[
        pltpu.VMEM((NCHUNK, CHUNK), jnp.int32),
        pltpu.VMEM((NCHUNK, CHUNK), jnp.int32),
        pltpu.VMEM((NCHUNK, CHUNK), jnp.int32),
        pltpu.VMEM((NCHUNK, CHUNK), jnp.int32),
        pltpu.VMEM((BPW,), jnp.int32),
        pltpu.VMEM((BPW,), jnp.int32),
        pltpu.VMEM((BPW, D), jnp.float32),
        pltpu.VMEM((BPW, D), jnp.float32),
        pltpu.VMEM((BPW, L), jnp.float32),
        pltpu.VMEM((BPW, L), jnp.float32),
        pltpu.VMEM((L, L), jnp.float32),
        pltpu.VMEM((BPW,), jnp.float32),
        pltpu.SemaphoreType.DMA,
    ],
    compiler_params=pltpu.CompilerParams(needs_layout_passes=False,
                                         use_tc_tiling_on_sc=False),
)(_mf_body)


@jax.jit
def kernel(user_id, item_id, user_embedding, user_bias, item_embedding,
           item_bias):
    uid = user_id.astype(jnp.int32)
    iid = item_id.astype(jnp.int32)
    ub16 = user_bias.reshape(-1)[:NIDS].reshape(NIDS // L, L)
    ib16 = item_bias.reshape(-1)[:NIDS].reshape(NIDS // L, L)
    return _mf_kernel(uid, iid, user_embedding, ub16, item_embedding, ib16)


# speedup vs baseline: 1.0008x; 1.0008x over previous
import functools

import jax
import jax.numpy as jnp
from jax import lax
from jax.experimental import pallas as pl
from jax.experimental.pallas import tpu as pltpu
from jax.experimental.pallas import tpu_sc as plsc

NC = 2
NS = 16
NW = NC * NS
L = 16

B = 16384
D = 32
BPW = B // NW
CHUNK = 128
NCHUNK = BPW // CHUNK
GROUPS = BPW // L


def _mf_body(uid_hbm, iid_hbm, uemb_hbm, ubf_hbm, iemb_hbm, ibf_hbm,
             out_hbm, uid_v, iid_v, urows, irows, ub, ib, mt, out_v, sem):
    wid = lax.axis_index("s") * NC + lax.axis_index("c")
    base = wid * BPW

    for c in range(NCHUNK):
        pltpu.sync_copy(uid_hbm.at[pl.ds(base + c * CHUNK, CHUNK)], uid_v.at[c])
        pltpu.sync_copy(iid_hbm.at[pl.ds(base + c * CHUNK, CHUNK)], iid_v.at[c])

    copies = []
    for c in range(NCHUNK):
        sl = pl.ds(c * CHUNK, CHUNK)
        copies.append(pltpu.async_copy(uemb_hbm.at[uid_v.at[c]], urows.at[sl], sem))
        copies.append(pltpu.async_copy(iemb_hbm.at[iid_v.at[c]], irows.at[sl], sem))
        copies.append(pltpu.async_copy(ubf_hbm.at[uid_v.at[c]], ub.at[sl], sem))
        copies.append(pltpu.async_copy(ibf_hbm.at[iid_v.at[c]], ib.at[sl], sem))
    for cp in copies:
        cp.wait()

    lanes = lax.iota(jnp.int32, L)

    def group(g, carry):
        r0 = pl.multiple_of(g * L, L)
        for r in range(L):
            row = r0 + r
            p0 = urows[row, pl.ds(0, L)]
            p1 = urows[row, pl.ds(L, L)]
            q0 = irows[row, pl.ds(0, L)]
            q1 = irows[row, pl.ds(L, L)]
            a = p0 * q0 + p1 * q1
            plsc.store_scatter(mt, [lanes, jnp.full((L,), r, jnp.int32)], a)
        acc = ub[pl.ds(r0, L)] + ib[pl.ds(r0, L)]
        for j in range(L):
            acc = acc + mt[j, pl.ds(0, L)]
        out_v[pl.ds(r0, L)] = acc
        return carry

    lax.fori_loop(0, GROUPS, group, 0)
    pltpu.sync_copy(out_v, out_hbm.at[pl.ds(base, BPW)])


_mf_kernel = functools.partial(
    pl.kernel,
    out_type=jax.ShapeDtypeStruct((B,), jnp.float32),
    mesh=plsc.VectorSubcoreMesh(
        core_axis_name="c", subcore_axis_name="s",
        num_cores=NC, num_subcores=NS),
    scratch_types=[
        pltpu.VMEM((NCHUNK, CHUNK), jnp.int32),
        pltpu.VMEM((NCHUNK, CHUNK), jnp.int32),
        pltpu.VMEM((BPW, D), jnp.float32),
        pltpu.VMEM((BPW, D), jnp.float32),
        pltpu.VMEM((BPW,), jnp.float32),
        pltpu.VMEM((BPW,), jnp.float32),
        pltpu.VMEM((L, L), jnp.float32),
        pltpu.VMEM((BPW,), jnp.float32),
        pltpu.SemaphoreType.DMA,
    ],
    compiler_params=pltpu.CompilerParams(needs_layout_passes=False,
                                         use_tc_tiling_on_sc=False),
)(_mf_body)


@jax.jit
def kernel(user_id, item_id, user_embedding, user_bias, item_embedding,
           item_bias):
    uid = user_id.astype(jnp.int32)
    iid = item_id.astype(jnp.int32)
    return _mf_kernel(uid, iid, user_embedding, user_bias.reshape(-1),
                      item_embedding, item_bias.reshape(-1))
